# baseline (speedup 1.0000x reference)
import jax
import jax.numpy as jnp
from jax.experimental import pallas as pl


def kernel(pred_edge_attr, edge_attr, edge_y, node_x, edge_index):
    src = edge_index[0]
    dst = edge_index[1]
    num_nodes = node_x.shape[0]
    error = (jax.ops.segment_sum(pred_edge_attr, src, num_segments=num_nodes)
             - jax.ops.segment_sum(pred_edge_attr, dst, num_segments=num_nodes))
    demands = node_x[:, 0]
    error = error - demands[:, None]
    layers = error.shape[1]
    discounting = 0.9
    exponents = jnp.arange(1, layers + 1, dtype=jnp.float32)[::-1]
    base = (jnp.ones(layers, dtype=jnp.float32) * discounting) ** exponents
    error = (error ** 2) * base
    return jnp.mean(error.sum(axis=1))


# speedup vs baseline: 18.5694x; 18.5694x over previous
import jax
import jax.numpy as jnp
from jax import lax
from jax.experimental import pallas as pl
from jax.experimental.pallas import tpu as pltpu
from jax.experimental.pallas import tpu_sc as plsc

N = 50000
E = 1600000
L = 4

NC = 2
NS = 16
NW = NC * NS

EPW = 50176
E_PAD = NW * EPW
CH = 3136
NCHUNK = EPW // CH
GRP = CH // 16

N_PAD = 50048

F_GRID = 17
F_COLS = N_PAD // F_GRID

DISC = [0.9 ** 4, 0.9 ** 3, 0.9 ** 2, 0.9]


def _chunk_compute(idx_sv, idx_dv, val_v, acc_s, acc_d):
    def grp_body(g, _):
        iv_s = idx_sv[pl.ds(g * 16, 16)]
        iv_d = idx_dv[pl.ds(g * 16, 16)]
        vv = val_v[pl.ds(g * 16, 16)]
        plsc.addupdate_scatter(acc_s, [iv_s], vv)
        plsc.addupdate_scatter(acc_d, [iv_d], vv)
        return ()

    lax.fori_loop(0, GRP, grp_body, ())


def _sc_segsum(src_hbm, dst_hbm, predt_hbm, zrow_hbm, out_hbm,
               isa, ida, va, isb, idb, vb, sem_a, sem_b, acc_s, acc_d):
    c = lax.axis_index("c")
    s = lax.axis_index("s")
    w = s * NC + c
    base_w = w * EPW

    for l in range(L):
        pltpu.sync_copy(zrow_hbm, acc_s)
        pltpu.sync_copy(zrow_hbm, acc_d)

        def dstep(k, _):
            b0 = base_w + (2 * k) * CH
            b1 = b0 + CH
            ga = [pltpu.async_copy(src_hbm.at[pl.ds(b0, CH)], isa, sem_a),
                  pltpu.async_copy(dst_hbm.at[pl.ds(b0, CH)], ida, sem_a),
                  pltpu.async_copy(predt_hbm.at[l, pl.ds(b0, CH)], va, sem_a)]
            gb = [pltpu.async_copy(src_hbm.at[pl.ds(b1, CH)], isb, sem_b),
                  pltpu.async_copy(dst_hbm.at[pl.ds(b1, CH)], idb, sem_b),
                  pltpu.async_copy(predt_hbm.at[l, pl.ds(b1, CH)], vb, sem_b)]
            for g in ga:
                g.wait()
            _chunk_compute(isa, ida, va, acc_s, acc_d)
            for g in gb:
                g.wait()
            _chunk_compute(isb, idb, vb, acc_s, acc_d)
            return ()

        lax.fori_loop(0, NCHUNK // 2, dstep, ())

        pltpu.sync_copy(acc_s, out_hbm.at[l, 0, w])
        pltpu.sync_copy(acc_d, out_hbm.at[l, 1, w])


@jax.jit
def _sc_call(src_flat, dst_flat, predt, zrow):
    mesh = plsc.VectorSubcoreMesh(
        core_axis_name="c", subcore_axis_name="s",
        num_cores=NC, num_subcores=NS)
    return pl.kernel(
        _sc_segsum,
        out_type=jax.ShapeDtypeStruct((L, 2, NW, N_PAD), jnp.float32),
        mesh=mesh,
        scratch_types=[
            pltpu.VMEM((CH,), jnp.int32),
            pltpu.VMEM((CH,), jnp.int32),
            pltpu.VMEM((CH,), jnp.float32),
            pltpu.VMEM((CH,), jnp.int32),
            pltpu.VMEM((CH,), jnp.int32),
            pltpu.VMEM((CH,), jnp.float32),
            pltpu.SemaphoreType.DMA,
            pltpu.SemaphoreType.DMA,
            pltpu.VMEM((N_PAD,), jnp.float32),
            pltpu.VMEM((N_PAD,), jnp.float32),
        ],
        compiler_params=pltpu.CompilerParams(
            use_tc_tiling_on_sc=False, needs_layout_passes=False),
    )(src_flat, dst_flat, predt, zrow)


def _finisher_body(s0, d0, s1, d1, s2, d2, s3, d3, dem, out_ref):
    i = pl.program_id(0)
    part = jnp.float32(0.0)
    for l, (sr, dr) in enumerate(((s0, d0), (s1, d1), (s2, d2), (s3, d3))):
        err = (jnp.sum(sr[...], axis=0, keepdims=True)
               - jnp.sum(dr[...], axis=0, keepdims=True)
               - dem[...])
        part = part + jnp.float32(DISC[l]) * jnp.sum(err * err)

    @pl.when(i == 0)
    def _init():
        out_ref[0, 0] = 0.0

    out_ref[0, 0] += part * (1.0 / N)


@jax.jit
def _finish(parts, dem):
    blk = pl.BlockSpec((NW, F_COLS), lambda i: (0, i))
    dblk = pl.BlockSpec((1, F_COLS), lambda i: (0, i))
    ins = [parts[l][d] for l in range(L) for d in range(2)]
    return pl.pallas_call(
        _finisher_body,
        grid=(F_GRID,),
        in_specs=[blk] * 8 + [dblk],
        out_specs=pl.BlockSpec((1, 1), lambda i: (0, 0),
                               memory_space=pltpu.SMEM),
        out_shape=jax.ShapeDtypeStruct((1, 1), jnp.float32),
    )(*ins, dem)


def kernel(pred_edge_attr, edge_attr, edge_y, node_x, edge_index):
    pad_e = E_PAD - E
    src_flat = jnp.concatenate([edge_index[0], jnp.zeros((pad_e,), jnp.int32)])
    dst_flat = jnp.concatenate([edge_index[1], jnp.zeros((pad_e,), jnp.int32)])
    predt = jnp.concatenate(
        [pred_edge_attr.T, jnp.zeros((L, pad_e), jnp.float32)], axis=1)
    zrow = jnp.zeros((N_PAD,), jnp.float32)

    out = _sc_call(src_flat, dst_flat, predt, zrow)

    demands = node_x[:, 0]
    dem = jnp.concatenate(
        [demands, jnp.zeros((N_PAD - N,), jnp.float32)]).reshape(1, N_PAD)
    parts = [[out[l, 0], out[l, 1]] for l in range(L)]
    loss = _finish(parts, dem)
    return loss[0, 0]
